# baseline (speedup 1.0000x reference)
import functools

import jax
import jax.numpy as jnp
from jax import lax
from jax.experimental import pallas as pl


def _gating_qkv_body(x_ref, sim_ref, gates_ref, wqkv_ref,
                     q_ref, k_ref, v_ref, w_ref, *, E, HD):
    x = x_ref[...]
    sim = sim_ref[...]
    g = gates_ref[...]

    rn = jnp.sqrt(jnp.sum(x * x, axis=1, keepdims=True))
    hn = x / jnp.maximum(rn, 1e-12)
    cn = jnp.sqrt(jnp.sum(sim * sim, axis=0, keepdims=True))
    sn = sim / jnp.maximum(cn, 1e-12)

    sig = 1.0 / (1.0 + jnp.exp(-g))
    logits = jnp.dot(hn, sn, preferred_element_type=jnp.float32) - sig
    gated = jnp.maximum(logits, 0.0)
    mask = (gated > 0.0).astype(jnp.float32)
    inactive = jnp.sum(mask, axis=1, keepdims=True) == 0.0

    BN = x.shape[0]
    eidx = lax.broadcasted_iota(jnp.int32, (BN, E), 1)
    m1 = jnp.max(logits, axis=1, keepdims=True)
    i1 = jnp.min(jnp.where(logits == m1, eidx, E), axis=1, keepdims=True)
    l2 = jnp.where(eidx == i1, -jnp.inf, logits)
    m2 = jnp.max(l2, axis=1, keepdims=True)
    i2 = jnp.min(jnp.where(l2 == m2, eidx, E), axis=1, keepdims=True)
    fb = (eidx == i1) | (eidx == i2)
    am = jnp.where(inactive & fb, 1.0, mask)

    gm = jnp.where(am > 0.0, gated, -1e9)
    gmax = jnp.max(gm, axis=1, keepdims=True)
    e = jnp.exp(gm - gmax)
    probs = e / jnp.sum(e, axis=1, keepdims=True)
    w_ref[...] = probs * am

    p = jnp.dot(x, wqkv_ref[...], preferred_element_type=jnp.float32)
    q = jnp.zeros((BN, HD), jnp.float32)
    k = jnp.zeros((BN, HD), jnp.float32)
    v = jnp.zeros((BN, HD), jnp.float32)
    for i in range(E):
        mi = am[:, i:i + 1]
        base = i * 3 * HD
        q = q + mi * p[:, base:base + HD]
        k = k + mi * p[:, base + HD:base + 2 * HD]
        v = v + mi * p[:, base + 2 * HD:base + 3 * HD]
    q_ref[...] = q
    k_ref[...] = k
    v_ref[...] = v


def _attn_out_body(q_ref, k_ref, v_ref, w_ref, o_ref, out_ref, *,
                   BQ, BK, T, E, HD, scale):
    qb = pl.program_id(1)
    q = q_ref[...]
    rows = qb * BQ + lax.broadcasted_iota(jnp.int32, (BQ, BK), 0)
    cols0 = lax.broadcasted_iota(jnp.int32, (BQ, BK), 1)

    def step(kb, carry):
        m, l, acc = carry
        kblk = k_ref[pl.ds(kb * BK, BK), :]
        vblk = v_ref[pl.ds(kb * BK, BK), :]
        s = lax.dot_general(q, kblk, (((1,), (1,)), ((), ())),
                            preferred_element_type=jnp.float32)
        s = jnp.where(kb * BK + cols0 <= rows, s * scale, -1e9)
        m_new = jnp.maximum(m, jnp.max(s, axis=1, keepdims=True))
        p = jnp.exp(s - m_new)
        alpha = jnp.exp(m - m_new)
        l = l * alpha + jnp.sum(p, axis=1, keepdims=True)
        acc = acc * alpha + jnp.dot(p, vblk, preferred_element_type=jnp.float32)
        return m_new, l, acc

    m0 = jnp.full((BQ, 1), -jnp.inf, jnp.float32)
    l0 = jnp.zeros((BQ, 1), jnp.float32)
    a0 = jnp.zeros((BQ, HD), jnp.float32)
    nkb = (qb + 1) * BQ // BK
    m, l, acc = lax.fori_loop(0, nkb, step, (m0, l0, a0))
    oh = acc / l

    w = w_ref[...]
    a2 = jnp.concatenate([oh * w[:, i:i + 1] for i in range(E)], axis=1)
    out_ref[...] = jnp.dot(a2, o_ref[...], preferred_element_type=jnp.float32)


def kernel(hidden_states, sim_matrix, gates, q_proj, k_proj, v_proj, o_proj):
    B, T, C = hidden_states.shape
    E = sim_matrix.shape[1]
    HD = q_proj.shape[2]
    N = B * T
    flat = hidden_states.reshape(N, C)

    wqkv = jnp.concatenate([q_proj, k_proj, v_proj], axis=2)
    wqkv = wqkv.transpose(1, 0, 2).reshape(C, E * 3 * HD)
    o_stack = o_proj.reshape(E * HD, C)
    gates_row = gates.reshape(1, E)

    BN = 1024 if N % 1024 == 0 else N
    g1 = N // BN
    q, k, v, w = pl.pallas_call(
        functools.partial(_gating_qkv_body, E=E, HD=HD),
        grid=(g1,),
        in_specs=[
            pl.BlockSpec((BN, C), lambda i: (i, 0)),
            pl.BlockSpec((C, E), lambda i: (0, 0)),
            pl.BlockSpec((1, E), lambda i: (0, 0)),
            pl.BlockSpec((C, E * 3 * HD), lambda i: (0, 0)),
        ],
        out_specs=[
            pl.BlockSpec((BN, HD), lambda i: (i, 0)),
            pl.BlockSpec((BN, HD), lambda i: (i, 0)),
            pl.BlockSpec((BN, HD), lambda i: (i, 0)),
            pl.BlockSpec((BN, E), lambda i: (i, 0)),
        ],
        out_shape=[
            jax.ShapeDtypeStruct((N, HD), jnp.float32),
            jax.ShapeDtypeStruct((N, HD), jnp.float32),
            jax.ShapeDtypeStruct((N, HD), jnp.float32),
            jax.ShapeDtypeStruct((N, E), jnp.float32),
        ],
    )(flat, sim_matrix, gates_row, wqkv)

    qb3 = q.reshape(B, T, HD)
    kb3 = k.reshape(B, T, HD)
    vb3 = v.reshape(B, T, HD)
    wb3 = w.reshape(B, T, E)

    BQ = 256 if T % 256 == 0 else T
    BK = BQ
    scale = 1.0 / float(HD) ** 0.5
    out = pl.pallas_call(
        functools.partial(_attn_out_body, BQ=BQ, BK=BK, T=T, E=E, HD=HD,
                          scale=scale),
        grid=(B, T // BQ),
        in_specs=[
            pl.BlockSpec((None, BQ, HD), lambda b, i: (b, i, 0)),
            pl.BlockSpec((None, T, HD), lambda b, i: (b, 0, 0)),
            pl.BlockSpec((None, T, HD), lambda b, i: (b, 0, 0)),
            pl.BlockSpec((None, BQ, E), lambda b, i: (b, i, 0)),
            pl.BlockSpec((E * HD, C), lambda b, i: (0, 0)),
        ],
        out_specs=pl.BlockSpec((None, BQ, C), lambda b, i: (b, i, 0)),
        out_shape=jax.ShapeDtypeStruct((B, T, C), jnp.float32),
    )(qb3, kb3, vb3, wb3, o_stack)
    return out


# speedup vs baseline: 1.1258x; 1.1258x over previous
import functools

import jax
import jax.numpy as jnp
from jax import lax
from jax.experimental import pallas as pl


def _gating_qkv_body(x_ref, sim_ref, gates_ref, wqkv_ref,
                     q_ref, k_ref, v_ref, w_ref, *, E, HD):
    x = x_ref[...]
    sim = sim_ref[...]
    g = gates_ref[...]

    rn = jnp.sqrt(jnp.sum(x * x, axis=1, keepdims=True))
    hn = x / jnp.maximum(rn, 1e-12)
    cn = jnp.sqrt(jnp.sum(sim * sim, axis=0, keepdims=True))
    sn = sim / jnp.maximum(cn, 1e-12)

    sig = 1.0 / (1.0 + jnp.exp(-g))
    logits = jnp.dot(hn, sn, preferred_element_type=jnp.float32) - sig
    gated = jnp.maximum(logits, 0.0)
    mask = (gated > 0.0).astype(jnp.float32)
    inactive = jnp.sum(mask, axis=1, keepdims=True) == 0.0

    BN = x.shape[0]
    eidx = lax.broadcasted_iota(jnp.int32, (BN, E), 1)
    m1 = jnp.max(logits, axis=1, keepdims=True)
    i1 = jnp.min(jnp.where(logits == m1, eidx, E), axis=1, keepdims=True)
    l2 = jnp.where(eidx == i1, -jnp.inf, logits)
    m2 = jnp.max(l2, axis=1, keepdims=True)
    i2 = jnp.min(jnp.where(l2 == m2, eidx, E), axis=1, keepdims=True)
    fb = (eidx == i1) | (eidx == i2)
    am = jnp.where(inactive & fb, 1.0, mask)

    gm = jnp.where(am > 0.0, gated, -1e9)
    gmax = jnp.max(gm, axis=1, keepdims=True)
    e = jnp.exp(gm - gmax)
    probs = e / jnp.sum(e, axis=1, keepdims=True)
    w_ref[...] = probs * am

    p = jnp.dot(x, wqkv_ref[...], preferred_element_type=jnp.float32)
    q = jnp.zeros((BN, HD), jnp.float32)
    k = jnp.zeros((BN, HD), jnp.float32)
    v = jnp.zeros((BN, HD), jnp.float32)
    for i in range(E):
        mi = am[:, i:i + 1]
        base = i * 3 * HD
        q = q + mi * p[:, base:base + HD]
        k = k + mi * p[:, base + HD:base + 2 * HD]
        v = v + mi * p[:, base + 2 * HD:base + 3 * HD]
    q_ref[...] = q
    k_ref[...] = k
    v_ref[...] = v


def _attn_out_body(q_ref, k_ref, v_ref, w_ref, o_ref, out_ref, *,
                   BQ, T, E, HD, scale):
    qb = pl.program_id(1)
    q = q_ref[...]
    k = k_ref[...]
    s = lax.dot_general(q, k, (((1,), (1,)), ((), ())),
                        preferred_element_type=jnp.float32)
    rows = qb * BQ + lax.broadcasted_iota(jnp.int32, (BQ, T), 0)
    cols = lax.broadcasted_iota(jnp.int32, (BQ, T), 1)
    s = jnp.where(cols <= rows, s * scale, -1e9)
    m = jnp.max(s, axis=1, keepdims=True)
    p = jnp.exp(s - m)
    a = p / jnp.sum(p, axis=1, keepdims=True)
    oh = jnp.dot(a, v_ref[...], preferred_element_type=jnp.float32)

    w = w_ref[...]
    a2 = jnp.concatenate([oh * w[:, i:i + 1] for i in range(E)], axis=1)
    out_ref[...] = jnp.dot(a2.astype(jnp.bfloat16), o_ref[...],
                           preferred_element_type=jnp.float32)


def kernel(hidden_states, sim_matrix, gates, q_proj, k_proj, v_proj, o_proj):
    B, T, C = hidden_states.shape
    E = sim_matrix.shape[1]
    HD = q_proj.shape[2]
    N = B * T
    flat = hidden_states.reshape(N, C)

    wqkv = jnp.concatenate([q_proj, k_proj, v_proj], axis=2)
    wqkv = wqkv.transpose(1, 0, 2).reshape(C, E * 3 * HD)
    o_stack = o_proj.reshape(E * HD, C)
    gates_row = gates.reshape(1, E)

    BN = 1024 if N % 1024 == 0 else N
    g1 = N // BN
    q, k, v, w = pl.pallas_call(
        functools.partial(_gating_qkv_body, E=E, HD=HD),
        grid=(g1,),
        in_specs=[
            pl.BlockSpec((BN, C), lambda i: (i, 0)),
            pl.BlockSpec((C, E), lambda i: (0, 0)),
            pl.BlockSpec((1, E), lambda i: (0, 0)),
            pl.BlockSpec((C, E * 3 * HD), lambda i: (0, 0)),
        ],
        out_specs=[
            pl.BlockSpec((BN, HD), lambda i: (i, 0)),
            pl.BlockSpec((BN, HD), lambda i: (i, 0)),
            pl.BlockSpec((BN, HD), lambda i: (i, 0)),
            pl.BlockSpec((BN, E), lambda i: (i, 0)),
        ],
        out_shape=[
            jax.ShapeDtypeStruct((N, HD), jnp.float32),
            jax.ShapeDtypeStruct((N, HD), jnp.float32),
            jax.ShapeDtypeStruct((N, HD), jnp.float32),
            jax.ShapeDtypeStruct((N, E), jnp.float32),
        ],
    )(flat, sim_matrix, gates_row, wqkv)

    qb3 = q.reshape(B, T, HD)
    kb3 = k.reshape(B, T, HD)
    vb3 = v.reshape(B, T, HD)
    wb3 = w.reshape(B, T, E)

    o_stack = o_stack.astype(jnp.bfloat16)
    BQ = 256 if T % 256 == 0 else T
    scale = 1.0 / float(HD) ** 0.5
    out = pl.pallas_call(
        functools.partial(_attn_out_body, BQ=BQ, T=T, E=E, HD=HD,
                          scale=scale),
        grid=(B, T // BQ),
        in_specs=[
            pl.BlockSpec((None, BQ, HD), lambda b, i: (b, i, 0)),
            pl.BlockSpec((None, T, HD), lambda b, i: (b, 0, 0)),
            pl.BlockSpec((None, T, HD), lambda b, i: (b, 0, 0)),
            pl.BlockSpec((None, BQ, E), lambda b, i: (b, i, 0)),
            pl.BlockSpec((E * HD, C), lambda b, i: (0, 0)),
        ],
        out_specs=pl.BlockSpec((None, BQ, C), lambda b, i: (b, i, 0)),
        out_shape=jax.ShapeDtypeStruct((B, T, C), jnp.float32),
    )(qb3, kb3, vb3, wb3, o_stack)
    return out


# speedup vs baseline: 1.2492x; 1.1096x over previous
import functools

import jax
import jax.numpy as jnp
from jax import lax
from jax.experimental import pallas as pl


def _gating_qkv_body(x_ref, sim_ref, gates_ref, wqkv_ref,
                     q_ref, k_ref, v_ref, w_ref, *, E, HD):
    x = x_ref[...]
    sim = sim_ref[...]
    g = gates_ref[...]

    rn = jnp.sqrt(jnp.sum(x * x, axis=1, keepdims=True))
    hn = x / jnp.maximum(rn, 1e-12)
    cn = jnp.sqrt(jnp.sum(sim * sim, axis=0, keepdims=True))
    sn = sim / jnp.maximum(cn, 1e-12)

    sig = 1.0 / (1.0 + jnp.exp(-g))
    logits = jnp.dot(hn, sn, preferred_element_type=jnp.float32) - sig
    gated = jnp.maximum(logits, 0.0)
    mask = (gated > 0.0).astype(jnp.float32)
    inactive = jnp.sum(mask, axis=1, keepdims=True) == 0.0

    BN = x.shape[0]
    eidx = lax.broadcasted_iota(jnp.int32, (BN, E), 1)
    m1 = jnp.max(logits, axis=1, keepdims=True)
    i1 = jnp.min(jnp.where(logits == m1, eidx, E), axis=1, keepdims=True)
    l2 = jnp.where(eidx == i1, -jnp.inf, logits)
    m2 = jnp.max(l2, axis=1, keepdims=True)
    i2 = jnp.min(jnp.where(l2 == m2, eidx, E), axis=1, keepdims=True)
    fb = (eidx == i1) | (eidx == i2)
    am = jnp.where(inactive & fb, 1.0, mask)

    gm = jnp.where(am > 0.0, gated, -1e9)
    gmax = jnp.max(gm, axis=1, keepdims=True)
    e = jnp.exp(gm - gmax)
    probs = e / jnp.sum(e, axis=1, keepdims=True)
    w_ref[...] = probs * am

    p = jnp.dot(x.astype(jnp.bfloat16), wqkv_ref[...],
                preferred_element_type=jnp.float32)
    q = jnp.zeros((BN, HD), jnp.float32)
    k = jnp.zeros((BN, HD), jnp.float32)
    v = jnp.zeros((BN, HD), jnp.float32)
    for i in range(E):
        mi = am[:, i:i + 1]
        base = i * 3 * HD
        q = q + mi * p[:, base:base + HD]
        k = k + mi * p[:, base + HD:base + 2 * HD]
        v = v + mi * p[:, base + 2 * HD:base + 3 * HD]
    q_ref[...] = q
    k_ref[...] = k
    v_ref[...] = v


def _attn_out_body(q_ref, k_ref, v_ref, w_ref, o_ref, out_ref, *,
                   BQ, T, E, HD, scale):
    qb = pl.program_id(1)
    q = q_ref[...]
    k = k_ref[...]
    s = lax.dot_general(q, k, (((1,), (1,)), ((), ())),
                        preferred_element_type=jnp.float32)
    rows = qb * BQ + lax.broadcasted_iota(jnp.int32, (BQ, T), 0)
    cols = lax.broadcasted_iota(jnp.int32, (BQ, T), 1)
    s = jnp.where(cols <= rows, s * scale, -1e9)
    m = jnp.max(s, axis=1, keepdims=True)
    p = jnp.exp(s - m)
    a = p / jnp.sum(p, axis=1, keepdims=True)
    oh = jnp.dot(a, v_ref[...], preferred_element_type=jnp.float32)

    w = w_ref[...]
    a2 = jnp.concatenate([oh * w[:, i:i + 1] for i in range(E)], axis=1)
    out_ref[...] = jnp.dot(a2.astype(jnp.bfloat16), o_ref[...],
                           preferred_element_type=jnp.float32)


def kernel(hidden_states, sim_matrix, gates, q_proj, k_proj, v_proj, o_proj):
    B, T, C = hidden_states.shape
    E = sim_matrix.shape[1]
    HD = q_proj.shape[2]
    N = B * T
    flat = hidden_states.reshape(N, C)

    wqkv = jnp.concatenate([q_proj, k_proj, v_proj], axis=2)
    wqkv = wqkv.transpose(1, 0, 2).reshape(C, E * 3 * HD).astype(jnp.bfloat16)
    o_stack = o_proj.reshape(E * HD, C)
    gates_row = gates.reshape(1, E)

    BN = 1024 if N % 1024 == 0 else N
    g1 = N // BN
    q, k, v, w = pl.pallas_call(
        functools.partial(_gating_qkv_body, E=E, HD=HD),
        grid=(g1,),
        in_specs=[
            pl.BlockSpec((BN, C), lambda i: (i, 0)),
            pl.BlockSpec((C, E), lambda i: (0, 0)),
            pl.BlockSpec((1, E), lambda i: (0, 0)),
            pl.BlockSpec((C, E * 3 * HD), lambda i: (0, 0)),
        ],
        out_specs=[
            pl.BlockSpec((BN, HD), lambda i: (i, 0)),
            pl.BlockSpec((BN, HD), lambda i: (i, 0)),
            pl.BlockSpec((BN, HD), lambda i: (i, 0)),
            pl.BlockSpec((BN, E), lambda i: (i, 0)),
        ],
        out_shape=[
            jax.ShapeDtypeStruct((N, HD), jnp.float32),
            jax.ShapeDtypeStruct((N, HD), jnp.float32),
            jax.ShapeDtypeStruct((N, HD), jnp.float32),
            jax.ShapeDtypeStruct((N, E), jnp.float32),
        ],
    )(flat, sim_matrix, gates_row, wqkv)

    qb3 = q.reshape(B, T, HD)
    kb3 = k.reshape(B, T, HD)
    vb3 = v.reshape(B, T, HD)
    wb3 = w.reshape(B, T, E)

    o_stack = o_stack.astype(jnp.bfloat16)
    BQ = 256 if T % 256 == 0 else T
    scale = 1.0 / float(HD) ** 0.5
    out = pl.pallas_call(
        functools.partial(_attn_out_body, BQ=BQ, T=T, E=E, HD=HD,
                          scale=scale),
        grid=(B, T // BQ),
        in_specs=[
            pl.BlockSpec((None, BQ, HD), lambda b, i: (b, i, 0)),
            pl.BlockSpec((None, T, HD), lambda b, i: (b, 0, 0)),
            pl.BlockSpec((None, T, HD), lambda b, i: (b, 0, 0)),
            pl.BlockSpec((None, BQ, E), lambda b, i: (b, i, 0)),
            pl.BlockSpec((E * HD, C), lambda b, i: (0, 0)),
        ],
        out_specs=pl.BlockSpec((None, BQ, C), lambda b, i: (b, i, 0)),
        out_shape=jax.ShapeDtypeStruct((B, T, C), jnp.float32),
    )(qb3, kb3, vb3, wb3, o_stack)
    return out


# speedup vs baseline: 1.2958x; 1.0374x over previous
import functools

import jax
import jax.numpy as jnp
from jax import lax
from jax.experimental import pallas as pl


def _gating_qkv_body(x_ref, sim_ref, gates_ref, wqkv_ref,
                     q_ref, k_ref, v_ref, w_ref, *, E, HD):
    x = x_ref[...]
    sim = sim_ref[...]
    g = gates_ref[...]

    rn = jnp.sqrt(jnp.sum(x * x, axis=1, keepdims=True))
    hn = x / jnp.maximum(rn, 1e-12)
    cn = jnp.sqrt(jnp.sum(sim * sim, axis=0, keepdims=True))
    sn = sim / jnp.maximum(cn, 1e-12)

    sig = 1.0 / (1.0 + jnp.exp(-g))
    logits = jnp.dot(hn, sn, preferred_element_type=jnp.float32) - sig
    gated = jnp.maximum(logits, 0.0)
    mask = (gated > 0.0).astype(jnp.float32)
    inactive = jnp.sum(mask, axis=1, keepdims=True) == 0.0

    BN = x.shape[0]
    eidx = lax.broadcasted_iota(jnp.int32, (BN, E), 1)
    m1 = jnp.max(logits, axis=1, keepdims=True)
    i1 = jnp.min(jnp.where(logits == m1, eidx, E), axis=1, keepdims=True)
    l2 = jnp.where(eidx == i1, -jnp.inf, logits)
    m2 = jnp.max(l2, axis=1, keepdims=True)
    i2 = jnp.min(jnp.where(l2 == m2, eidx, E), axis=1, keepdims=True)
    fb = (eidx == i1) | (eidx == i2)
    am = jnp.where(inactive & fb, 1.0, mask)

    gm = jnp.where(am > 0.0, gated, -1e9)
    gmax = jnp.max(gm, axis=1, keepdims=True)
    e = jnp.exp(gm - gmax)
    probs = e / jnp.sum(e, axis=1, keepdims=True)
    w_ref[...] = probs * am

    p = jnp.dot(x.astype(jnp.bfloat16), wqkv_ref[...],
                preferred_element_type=jnp.float32)
    q = jnp.zeros((BN, HD), jnp.float32)
    k = jnp.zeros((BN, HD), jnp.float32)
    v = jnp.zeros((BN, HD), jnp.float32)
    for i in range(E):
        mi = am[:, i:i + 1]
        base = i * 3 * HD
        q = q + mi * p[:, base:base + HD]
        k = k + mi * p[:, base + HD:base + 2 * HD]
        v = v + mi * p[:, base + 2 * HD:base + 3 * HD]
    q_ref[...] = q
    k_ref[...] = k
    v_ref[...] = v


def _attn_out_body(q_ref, k_ref, v_ref, w_ref, o_ref, out_ref, *,
                   BQ, T, E, HD, scale):
    qb = pl.program_id(1)
    q = q_ref[...].astype(jnp.bfloat16)
    k = k_ref[...].astype(jnp.bfloat16)
    s = lax.dot_general(q, k, (((1,), (1,)), ((), ())),
                        preferred_element_type=jnp.float32)
    rows = qb * BQ + lax.broadcasted_iota(jnp.int32, (BQ, T), 0)
    cols = lax.broadcasted_iota(jnp.int32, (BQ, T), 1)
    s = jnp.where(cols <= rows, s * scale, -1e9)
    m = jnp.max(s, axis=1, keepdims=True)
    p = jnp.exp(s - m)
    l = jnp.sum(p, axis=1, keepdims=True)
    oh = jnp.dot(p.astype(jnp.bfloat16), v_ref[...].astype(jnp.bfloat16),
                 preferred_element_type=jnp.float32)
    oh = oh / l

    w = w_ref[...]
    a2 = jnp.concatenate([oh * w[:, i:i + 1] for i in range(E)], axis=1)
    out_ref[...] = jnp.dot(a2.astype(jnp.bfloat16), o_ref[...],
                           preferred_element_type=jnp.float32)


def kernel(hidden_states, sim_matrix, gates, q_proj, k_proj, v_proj, o_proj):
    B, T, C = hidden_states.shape
    E = sim_matrix.shape[1]
    HD = q_proj.shape[2]
    N = B * T
    flat = hidden_states.reshape(N, C)

    wqkv = jnp.concatenate([q_proj, k_proj, v_proj], axis=2)
    wqkv = wqkv.transpose(1, 0, 2).reshape(C, E * 3 * HD).astype(jnp.bfloat16)
    o_stack = o_proj.reshape(E * HD, C)
    gates_row = gates.reshape(1, E)

    BN = 1024 if N % 1024 == 0 else N
    g1 = N // BN
    q, k, v, w = pl.pallas_call(
        functools.partial(_gating_qkv_body, E=E, HD=HD),
        grid=(g1,),
        in_specs=[
            pl.BlockSpec((BN, C), lambda i: (i, 0)),
            pl.BlockSpec((C, E), lambda i: (0, 0)),
            pl.BlockSpec((1, E), lambda i: (0, 0)),
            pl.BlockSpec((C, E * 3 * HD), lambda i: (0, 0)),
        ],
        out_specs=[
            pl.BlockSpec((BN, HD), lambda i: (i, 0)),
            pl.BlockSpec((BN, HD), lambda i: (i, 0)),
            pl.BlockSpec((BN, HD), lambda i: (i, 0)),
            pl.BlockSpec((BN, E), lambda i: (i, 0)),
        ],
        out_shape=[
            jax.ShapeDtypeStruct((N, HD), jnp.float32),
            jax.ShapeDtypeStruct((N, HD), jnp.float32),
            jax.ShapeDtypeStruct((N, HD), jnp.float32),
            jax.ShapeDtypeStruct((N, E), jnp.float32),
        ],
    )(flat, sim_matrix, gates_row, wqkv)

    qb3 = q.reshape(B, T, HD)
    kb3 = k.reshape(B, T, HD)
    vb3 = v.reshape(B, T, HD)
    wb3 = w.reshape(B, T, E)

    o_stack = o_stack.astype(jnp.bfloat16)
    BQ = 256 if T % 256 == 0 else T
    scale = 1.0 / float(HD) ** 0.5
    out = pl.pallas_call(
        functools.partial(_attn_out_body, BQ=BQ, T=T, E=E, HD=HD,
                          scale=scale),
        grid=(B, T // BQ),
        in_specs=[
            pl.BlockSpec((None, BQ, HD), lambda b, i: (b, i, 0)),
            pl.BlockSpec((None, T, HD), lambda b, i: (b, 0, 0)),
            pl.BlockSpec((None, T, HD), lambda b, i: (b, 0, 0)),
            pl.BlockSpec((None, BQ, E), lambda b, i: (b, i, 0)),
            pl.BlockSpec((E * HD, C), lambda b, i: (0, 0)),
        ],
        out_specs=pl.BlockSpec((None, BQ, C), lambda b, i: (b, i, 0)),
        out_shape=jax.ShapeDtypeStruct((B, T, C), jnp.float32),
    )(qb3, kb3, vb3, wb3, o_stack)
    return out
